# Initial kernel scaffold; baseline (speedup 1.0000x reference)
import functools

import jax
import jax.numpy as jnp
from jax import lax
from jax.experimental import pallas as pl
from jax.experimental.pallas import tpu as pltpu
from jax.experimental.pallas import tpu_sc as plsc

N = 10000
E = 320000
D_IN = 128
DH = 16

NC = 2
NS = 16
NW = NC * NS
CH = 128
J = 80
E_PAD = NW * J * CH
N_PAD = 10112
RPT = N_PAD // NS

_mesh = plsc.VectorSubcoreMesh(core_axis_name="c", subcore_axis_name="s")


@functools.partial(
    pl.kernel,
    mesh=_mesh,
    out_type=jax.ShapeDtypeStruct((NC * N_PAD,), jnp.float32),
    scratch_types=[
        pltpu.VMEM((J, CH), jnp.int32),
        pltpu.VMEM((CH,), jnp.float32),
        pltpu.VMEM((RPT,), jnp.float32),
    ],
)
def _deg_kernel(dst_hbm, ones_hbm, zeros_hbm, out_hbm, dst_v, ones_v, z_v):
    c = lax.axis_index("c")
    s = lax.axis_index("s")
    wid = s * NC + c

    def body(acc):
        pltpu.sync_copy(zeros_hbm, z_v)
        pltpu.sync_copy(z_v, acc.at[pl.ds(s * RPT, RPT)])
        pltpu.sync_copy(ones_hbm, ones_v)
        pltpu.sync_copy(dst_hbm.at[wid], dst_v)
        plsc.subcore_barrier()

        def step(j, _):
            pltpu.sync_copy(ones_v, acc.at[dst_v.at[j]], add=True)
            return ()

        lax.fori_loop(0, J, step, ())
        plsc.subcore_barrier()
        pltpu.sync_copy(acc.at[pl.ds(s * RPT, RPT)],
                        out_hbm.at[pl.ds(c * N_PAD + s * RPT, RPT)])

    pl.run_scoped(body, pltpu.VMEM_SHARED((N_PAD,), jnp.float32))


@functools.partial(
    pl.kernel,
    mesh=_mesh,
    out_type=jax.ShapeDtypeStruct((NC * N_PAD, DH), jnp.float32),
    scratch_types=[
        pltpu.VMEM((J, CH), jnp.int32),
        pltpu.VMEM((J, CH), jnp.int32),
        pltpu.VMEM((CH, DH), jnp.float32),
        pltpu.VMEM((RPT, DH), jnp.float32),
        pltpu.SemaphoreType.DMA,
    ],
)
def _agg_kernel(h_hbm, src_hbm, dst_hbm, zrows_hbm, out_hbm,
                src_v, dst_v, rows_v, z_v, sem):
    c = lax.axis_index("c")
    s = lax.axis_index("s")
    wid = s * NC + c

    def body(acc):
        pltpu.sync_copy(zrows_hbm, z_v)
        pltpu.sync_copy(z_v, acc.at[pl.ds(s * RPT, RPT)])
        pltpu.sync_copy(src_hbm.at[wid], src_v)
        pltpu.sync_copy(dst_hbm.at[wid], dst_v)
        plsc.subcore_barrier()

        def step(j, _):
            pltpu.async_copy(h_hbm.at[src_v.at[j]], rows_v, sem).wait()
            pltpu.sync_copy(rows_v, acc.at[dst_v.at[j]], add=True)
            return ()

        lax.fori_loop(0, J, step, ())
        plsc.subcore_barrier()
        pltpu.sync_copy(acc.at[pl.ds(s * RPT, RPT)],
                        out_hbm.at[pl.ds(c * N_PAD + s * RPT, RPT)])

    pl.run_scoped(body, pltpu.VMEM_SHARED((N_PAD, DH), jnp.float32))


def _in_body(degp_ref, x_ref, w_ref, h_ref, dinv_ref):
    deg = degp_ref[0] + degp_ref[1] + 1.0
    dinv = lax.rsqrt(deg)
    h = jnp.dot(x_ref[...], w_ref[...], preferred_element_type=jnp.float32)
    h_ref[0:N, :] = h * dinv
    h_ref[N:N_PAD, :] = jnp.zeros((N_PAD - N, DH), jnp.float32)
    dinv_ref[...] = dinv


def _mid_body(aggp_ref, h1_ref, dinv_ref, b_ref, w_ref, h_ref):
    su = aggp_ref[0, 0:N, :] + aggp_ref[1, 0:N, :] + h1_ref[0:N, :]
    z = jnp.maximum(su * dinv_ref[...] + b_ref[...], 0.0)
    h2 = jnp.dot(z, w_ref[...], preferred_element_type=jnp.float32)
    h_ref[0:N, :] = h2 * dinv_ref[...]
    h_ref[N:N_PAD, :] = jnp.zeros((N_PAD - N, DH), jnp.float32)


def _out_body(aggp_ref, h2_ref, dinv_ref, b_ref, out_ref):
    su = aggp_ref[0, 0:N, :] + aggp_ref[1, 0:N, :] + h2_ref[0:N, :]
    t = su * dinv_ref[...] + b_ref[...]
    m = jnp.max(t, axis=1, keepdims=True)
    lse = m + jnp.log(jnp.sum(jnp.exp(t - m), axis=1, keepdims=True))
    out_ref[...] = t - lse


_in_call = pl.pallas_call(
    _in_body,
    out_shape=[
        jax.ShapeDtypeStruct((N_PAD, DH), jnp.float32),
        jax.ShapeDtypeStruct((N, 1), jnp.float32),
    ],
)

_mid_call = pl.pallas_call(
    _mid_body,
    out_shape=jax.ShapeDtypeStruct((N_PAD, DH), jnp.float32),
)

_out_call = pl.pallas_call(
    _out_body,
    out_shape=jax.ShapeDtypeStruct((N, DH), jnp.float32),
)


@jax.jit
def kernel(x, edge_index, W1, b1, W2, b2):
    ei = edge_index.astype(jnp.int32)
    pad = jnp.full((E_PAD - E,), N, jnp.int32)
    src = jnp.concatenate([ei[0], pad]).reshape(NW, J, CH)
    dst = jnp.concatenate([ei[1], pad]).reshape(NW, J, CH)

    ones_c = jnp.ones((CH,), jnp.float32)
    zeros_r = jnp.zeros((RPT,), jnp.float32)
    zrows = jnp.zeros((RPT, DH), jnp.float32)

    degp = _deg_kernel(dst, ones_c, zeros_r)
    degp = degp.reshape(NC, N_PAD)[:, :N, None]

    h1p, dinv = _in_call(degp, x, W1)
    agg1 = _agg_kernel(h1p, src, dst, zrows).reshape(NC, N_PAD, DH)
    h2p = _mid_call(agg1, h1p, dinv, b1[None, :], W2)
    agg2 = _agg_kernel(h2p, src, dst, zrows).reshape(NC, N_PAD, DH)
    return _out_call(agg2, h2p, dinv, b2[None, :])


# speedup vs baseline: 29.5840x; 29.5840x over previous
import functools

import jax
import jax.numpy as jnp
from jax import lax
from jax.experimental import pallas as pl
from jax.experimental.pallas import tpu as pltpu
from jax.experimental.pallas import tpu_sc as plsc

N = 10000
E = 320000
D_IN = 128
DH = 16

NC = 2
NS = 16
NW = NC * NS
CH = 128
J = 80
E_PAD = NW * J * CH
N_PAD = 10112
RPT = N_PAD // NS


def _deg_body(dst_hbm, ones_hbm, zeros_hbm, out_hbm, dst_v, ones_v, z_v, acc):
    c = lax.axis_index("c")
    s = lax.axis_index("s")
    wid = s * NC + c

    pltpu.sync_copy(zeros_hbm, z_v)
    pltpu.sync_copy(z_v, acc.at[pl.ds(s * RPT, RPT)])
    pltpu.sync_copy(ones_hbm, ones_v)
    pltpu.sync_copy(dst_hbm.at[wid], dst_v)
    plsc.subcore_barrier()

    def step(j, _):
        pltpu.sync_copy(ones_v, acc.at[dst_v.at[j]], add=True)
        return ()

    lax.fori_loop(0, J, step, ())
    plsc.subcore_barrier()
    pltpu.sync_copy(acc.at[pl.ds(s * RPT, RPT)], z_v)
    pltpu.sync_copy(z_v, out_hbm.at[pl.ds(c * N_PAD + s * RPT, RPT)])


def _agg_body(h_hbm, src_hbm, dst_hbm, zrows_hbm, out_hbm,
              src_v, dst_v, rows_v, z_v, sem, acc):
    c = lax.axis_index("c")
    s = lax.axis_index("s")
    wid = s * NC + c

    pltpu.sync_copy(zrows_hbm, z_v)
    pltpu.sync_copy(z_v, acc.at[pl.ds(s * RPT, RPT)])
    pltpu.sync_copy(src_hbm.at[wid], src_v)
    pltpu.sync_copy(dst_hbm.at[wid], dst_v)
    plsc.subcore_barrier()

    def step(j, _):
        pltpu.async_copy(h_hbm.at[src_v.at[j]], rows_v, sem).wait()
        pltpu.sync_copy(rows_v, acc.at[dst_v.at[j]], add=True)
        return ()

    lax.fori_loop(0, J, step, ())
    plsc.subcore_barrier()
    pltpu.sync_copy(acc.at[pl.ds(s * RPT, RPT)], z_v)
    pltpu.sync_copy(z_v, out_hbm.at[pl.ds(c * N_PAD + s * RPT, RPT)])


@functools.cache
def _sc_kernels():
    mesh = plsc.VectorSubcoreMesh(core_axis_name="c", subcore_axis_name="s")
    params = pltpu.CompilerParams(use_tc_tiling_on_sc=False)
    deg = pl.kernel(
        _deg_body,
        mesh=mesh,
        compiler_params=params,
        out_type=jax.ShapeDtypeStruct((NC * N_PAD,), jnp.float32),
        scratch_types=[
            pltpu.VMEM((J, CH), jnp.int32),
            pltpu.VMEM((CH,), jnp.float32),
            pltpu.VMEM((RPT,), jnp.float32),
            pltpu.VMEM_SHARED((N_PAD,), jnp.float32),
        ],
    )
    agg = pl.kernel(
        _agg_body,
        mesh=mesh,
        compiler_params=params,
        out_type=jax.ShapeDtypeStruct((NC * N_PAD, DH), jnp.float32),
        scratch_types=[
            pltpu.VMEM((J, CH), jnp.int32),
            pltpu.VMEM((J, CH), jnp.int32),
            pltpu.VMEM((CH, DH), jnp.float32),
            pltpu.VMEM((RPT, DH), jnp.float32),
            pltpu.SemaphoreType.DMA,
            pltpu.VMEM_SHARED((N_PAD, DH), jnp.float32),
        ],
    )
    return deg, agg


def _in_body(degp_ref, x_ref, w_ref, h_ref, dinv_ref):
    deg = degp_ref[0] + degp_ref[1] + 1.0
    dinv = lax.rsqrt(deg)
    h = jnp.dot(x_ref[...], w_ref[...], preferred_element_type=jnp.float32)
    h_ref[0:N, :] = h * dinv
    h_ref[N:N_PAD, :] = jnp.zeros((N_PAD - N, DH), jnp.float32)
    dinv_ref[...] = dinv


def _mid_body(aggp_ref, h1_ref, dinv_ref, b_ref, w_ref, h_ref):
    su = aggp_ref[0, 0:N, :] + aggp_ref[1, 0:N, :] + h1_ref[0:N, :]
    z = jnp.maximum(su * dinv_ref[...] + b_ref[...], 0.0)
    h2 = jnp.dot(z, w_ref[...], preferred_element_type=jnp.float32)
    h_ref[0:N, :] = h2 * dinv_ref[...]
    h_ref[N:N_PAD, :] = jnp.zeros((N_PAD - N, DH), jnp.float32)


def _out_body(aggp_ref, h2_ref, dinv_ref, b_ref, out_ref):
    su = aggp_ref[0, 0:N, :] + aggp_ref[1, 0:N, :] + h2_ref[0:N, :]
    t = su * dinv_ref[...] + b_ref[...]
    m = jnp.max(t, axis=1, keepdims=True)
    lse = m + jnp.log(jnp.sum(jnp.exp(t - m), axis=1, keepdims=True))
    out_ref[...] = t - lse


_in_call = pl.pallas_call(
    _in_body,
    out_shape=[
        jax.ShapeDtypeStruct((N_PAD, DH), jnp.float32),
        jax.ShapeDtypeStruct((N, 1), jnp.float32),
    ],
)

_mid_call = pl.pallas_call(
    _mid_body,
    out_shape=jax.ShapeDtypeStruct((N_PAD, DH), jnp.float32),
)

_out_call = pl.pallas_call(
    _out_body,
    out_shape=jax.ShapeDtypeStruct((N, DH), jnp.float32),
)


@jax.jit
def kernel(x, edge_index, W1, b1, W2, b2):
    ei = edge_index.astype(jnp.int32)
    pad = jnp.full((E_PAD - E,), N, jnp.int32)
    src = jnp.concatenate([ei[0], pad]).reshape(NW, J, CH)
    dst = jnp.concatenate([ei[1], pad]).reshape(NW, J, CH)

    ones_c = jnp.ones((CH,), jnp.float32)
    zeros_r = jnp.zeros((RPT,), jnp.float32)
    zrows = jnp.zeros((RPT, DH), jnp.float32)

    deg_kernel, agg_kernel = _sc_kernels()
    degp = deg_kernel(dst, ones_c, zeros_r)
    degp = degp.reshape(NC, N_PAD)[:, :N, None]

    h1p, dinv = _in_call(degp, x, W1)
    agg1 = agg_kernel(h1p, src, dst, zrows).reshape(NC, N_PAD, DH)
    h2p = _mid_call(agg1, h1p, dinv, b1[None, :], W2)
    agg2 = agg_kernel(h2p, src, dst, zrows).reshape(NC, N_PAD, DH)
    return _out_call(agg2, h2p, dinv, b2[None, :])


# speedup vs baseline: 35.9915x; 1.2166x over previous
import functools

import jax
import jax.numpy as jnp
from jax import lax
from jax.experimental import pallas as pl
from jax.experimental.pallas import tpu as pltpu
from jax.experimental.pallas import tpu_sc as plsc

N = 10000
E = 320000
D_IN = 128
DH = 16

NC = 2
NS = 16
NW = NC * NS
CH = 128
J = 80
KB = 8
E_PAD = NW * J * CH
N_PAD = 10112
RPT = N_PAD // NS


def _deg_body(dst_hbm, ones_hbm, zeros_hbm, out_hbm, dst_v, ones_v, z_v, acc):
    c = lax.axis_index("c")
    s = lax.axis_index("s")
    wid = s * NC + c

    pltpu.sync_copy(zeros_hbm, z_v)
    pltpu.sync_copy(z_v, acc.at[pl.ds(s * RPT, RPT)])
    pltpu.sync_copy(ones_hbm, ones_v)
    pltpu.sync_copy(dst_hbm.at[wid], dst_v)
    plsc.subcore_barrier()

    def step(j, _):
        pltpu.sync_copy(ones_v, acc.at[dst_v.at[j]], add=True)
        return ()

    lax.fori_loop(0, J, step, ())
    plsc.subcore_barrier()
    pltpu.sync_copy(acc.at[pl.ds(s * RPT, RPT)], z_v)
    pltpu.sync_copy(z_v, out_hbm.at[pl.ds(c * N_PAD + s * RPT, RPT)])


def _agg_body(h_hbm, src_hbm, dst_hbm, zrows_hbm, out_hbm,
              src_v, dst_v, rows_v, z_v, gsem, ssem, acc):
    c = lax.axis_index("c")
    s = lax.axis_index("s")
    wid = s * NC + c

    pltpu.sync_copy(zrows_hbm, z_v)
    pltpu.sync_copy(z_v, acc.at[pl.ds(s * RPT, RPT)])
    pltpu.sync_copy(src_hbm.at[wid], src_v)
    pltpu.sync_copy(dst_hbm.at[wid], dst_v)
    plsc.subcore_barrier()

    def step(k, _):
        base = k * KB
        g = [pltpu.async_copy(h_hbm.at[src_v.at[base + b]],
                              rows_v.at[b], gsem)
             for b in range(KB)]
        for b in range(KB):
            g[b].wait()
        sc = [pltpu.async_copy(rows_v.at[b], acc.at[dst_v.at[base + b]],
                               ssem, add=True)
              for b in range(KB)]
        for b in range(KB):
            sc[b].wait()
        return ()

    lax.fori_loop(0, J // KB, step, ())
    plsc.subcore_barrier()
    pltpu.sync_copy(acc.at[pl.ds(s * RPT, RPT)], z_v)
    pltpu.sync_copy(z_v, out_hbm.at[pl.ds(c * N_PAD + s * RPT, RPT)])


@functools.cache
def _sc_kernels():
    mesh = plsc.VectorSubcoreMesh(core_axis_name="c", subcore_axis_name="s")
    params = pltpu.CompilerParams(use_tc_tiling_on_sc=False)
    deg = pl.kernel(
        _deg_body,
        mesh=mesh,
        compiler_params=params,
        out_type=jax.ShapeDtypeStruct((NC * N_PAD,), jnp.float32),
        scratch_types=[
            pltpu.VMEM((J, CH), jnp.int32),
            pltpu.VMEM((CH,), jnp.float32),
            pltpu.VMEM((RPT,), jnp.float32),
            pltpu.VMEM_SHARED((N_PAD,), jnp.float32),
        ],
    )
    agg = pl.kernel(
        _agg_body,
        mesh=mesh,
        compiler_params=params,
        out_type=jax.ShapeDtypeStruct((NC * N_PAD, DH), jnp.float32),
        scratch_types=[
            pltpu.VMEM((J, CH), jnp.int32),
            pltpu.VMEM((J, CH), jnp.int32),
            pltpu.VMEM((KB, CH, DH), jnp.float32),
            pltpu.VMEM((RPT, DH), jnp.float32),
            pltpu.SemaphoreType.DMA,
            pltpu.SemaphoreType.DMA,
            pltpu.VMEM_SHARED((N_PAD, DH), jnp.float32),
        ],
    )
    return deg, agg


def _in_body(degp_ref, x_ref, w_ref, h_ref, dinv_ref):
    deg = degp_ref[0] + degp_ref[1] + 1.0
    dinv = lax.rsqrt(deg)
    h = jnp.dot(x_ref[...], w_ref[...], preferred_element_type=jnp.float32)
    h_ref[0:N, :] = h * dinv
    h_ref[N:N_PAD, :] = jnp.zeros((N_PAD - N, DH), jnp.float32)
    dinv_ref[...] = dinv


def _mid_body(aggp_ref, h1_ref, dinv_ref, b_ref, w_ref, h_ref):
    su = aggp_ref[0, 0:N, :] + aggp_ref[1, 0:N, :] + h1_ref[0:N, :]
    z = jnp.maximum(su * dinv_ref[...] + b_ref[...], 0.0)
    h2 = jnp.dot(z, w_ref[...], preferred_element_type=jnp.float32)
    h_ref[0:N, :] = h2 * dinv_ref[...]
    h_ref[N:N_PAD, :] = jnp.zeros((N_PAD - N, DH), jnp.float32)


def _out_body(aggp_ref, h2_ref, dinv_ref, b_ref, out_ref):
    su = aggp_ref[0, 0:N, :] + aggp_ref[1, 0:N, :] + h2_ref[0:N, :]
    t = su * dinv_ref[...] + b_ref[...]
    m = jnp.max(t, axis=1, keepdims=True)
    lse = m + jnp.log(jnp.sum(jnp.exp(t - m), axis=1, keepdims=True))
    out_ref[...] = t - lse


_in_call = pl.pallas_call(
    _in_body,
    out_shape=[
        jax.ShapeDtypeStruct((N_PAD, DH), jnp.float32),
        jax.ShapeDtypeStruct((N, 1), jnp.float32),
    ],
)

_mid_call = pl.pallas_call(
    _mid_body,
    out_shape=jax.ShapeDtypeStruct((N_PAD, DH), jnp.float32),
)

_out_call = pl.pallas_call(
    _out_body,
    out_shape=jax.ShapeDtypeStruct((N, DH), jnp.float32),
)


@jax.jit
def kernel(x, edge_index, W1, b1, W2, b2):
    ei = edge_index.astype(jnp.int32)
    pad = jnp.full((E_PAD - E,), N, jnp.int32)
    src = jnp.concatenate([ei[0], pad]).reshape(NW, J, CH)
    dst = jnp.concatenate([ei[1], pad]).reshape(NW, J, CH)

    ones_c = jnp.ones((CH,), jnp.float32)
    zeros_r = jnp.zeros((RPT,), jnp.float32)
    zrows = jnp.zeros((RPT, DH), jnp.float32)

    deg_kernel, agg_kernel = _sc_kernels()
    degp = deg_kernel(dst, ones_c, zeros_r)
    degp = degp.reshape(NC, N_PAD)[:, :N, None]

    h1p, dinv = _in_call(degp, x, W1)
    agg1 = agg_kernel(h1p, src, dst, zrows).reshape(NC, N_PAD, DH)
    h2p = _mid_call(agg1, h1p, dinv, b1[None, :], W2)
    agg2 = agg_kernel(h2p, src, dst, zrows).reshape(NC, N_PAD, DH)
    return _out_call(agg2, h2p, dinv, b2[None, :])


# speedup vs baseline: 37.3133x; 1.0367x over previous
import functools

import jax
import jax.numpy as jnp
from jax import lax
from jax.experimental import pallas as pl
from jax.experimental.pallas import tpu as pltpu
from jax.experimental.pallas import tpu_sc as plsc

N = 10000
E = 320000
D_IN = 128
DH = 16

NC = 2
NS = 16
NW = NC * NS
CH = 128
J = 80
KB = 4
E_PAD = NW * J * CH
N_PAD = 10112
RPT = N_PAD // NS


def _deg_body(dst_hbm, ones_hbm, zeros_hbm, out_hbm, dst_v, ones_v, z_v, acc):
    c = lax.axis_index("c")
    s = lax.axis_index("s")
    wid = s * NC + c

    pltpu.sync_copy(zeros_hbm, z_v)
    pltpu.sync_copy(z_v, acc.at[pl.ds(s * RPT, RPT)])
    pltpu.sync_copy(ones_hbm, ones_v)
    pltpu.sync_copy(dst_hbm.at[wid], dst_v)
    plsc.subcore_barrier()

    def step(j, _):
        pltpu.sync_copy(ones_v, acc.at[dst_v.at[j]], add=True)
        return ()

    lax.fori_loop(0, J, step, ())
    plsc.subcore_barrier()
    pltpu.sync_copy(acc.at[pl.ds(s * RPT, RPT)], z_v)
    pltpu.sync_copy(z_v, out_hbm.at[pl.ds(c * N_PAD + s * RPT, RPT)])


def _agg_body(h_hbm, src_hbm, dst_hbm, zrows_hbm, out_hbm,
              src_v, dst_v, rows_v, z_v, gsem, ssemA, ssemB, acc):
    c = lax.axis_index("c")
    s = lax.axis_index("s")
    wid = s * NC + c

    pltpu.sync_copy(zrows_hbm, z_v)
    pltpu.sync_copy(z_v, acc.at[pl.ds(s * RPT, RPT)])
    pltpu.sync_copy(src_hbm.at[wid], src_v)
    pltpu.sync_copy(dst_hbm.at[wid], dst_v)
    plsc.subcore_barrier()

    def drain(set_idx, sem, base):
        for b in range(KB):
            pltpu.make_async_copy(
                rows_v.at[set_idx, b], acc.at[dst_v.at[base + b]], sem
            ).wait()

    def half(set_idx, sem, base):
        g = [pltpu.async_copy(h_hbm.at[src_v.at[base + b]],
                              rows_v.at[set_idx, b], gsem)
             for b in range(KB)]
        for b in range(KB):
            g[b].wait()
        for b in range(KB):
            pltpu.async_copy(rows_v.at[set_idx, b],
                             acc.at[dst_v.at[base + b]], sem, add=True)

    def step(k, _):
        base = k * 2 * KB

        @pl.when(k > 0)
        def _():
            drain(0, ssemA, base - 2 * KB)

        half(0, ssemA, base)

        @pl.when(k > 0)
        def _():
            drain(1, ssemB, base - KB)

        half(1, ssemB, base + KB)
        return ()

    lax.fori_loop(0, J // (2 * KB), step, ())
    drain(0, ssemA, J - 2 * KB)
    drain(1, ssemB, J - KB)
    plsc.subcore_barrier()
    pltpu.sync_copy(acc.at[pl.ds(s * RPT, RPT)], z_v)
    pltpu.sync_copy(z_v, out_hbm.at[pl.ds(c * N_PAD + s * RPT, RPT)])


@functools.cache
def _sc_kernels():
    mesh = plsc.VectorSubcoreMesh(core_axis_name="c", subcore_axis_name="s")
    params = pltpu.CompilerParams(use_tc_tiling_on_sc=False)
    deg = pl.kernel(
        _deg_body,
        mesh=mesh,
        compiler_params=params,
        out_type=jax.ShapeDtypeStruct((NC * N_PAD,), jnp.float32),
        scratch_types=[
            pltpu.VMEM((J, CH), jnp.int32),
            pltpu.VMEM((CH,), jnp.float32),
            pltpu.VMEM((RPT,), jnp.float32),
            pltpu.VMEM_SHARED((N_PAD,), jnp.float32),
        ],
    )
    agg = pl.kernel(
        _agg_body,
        mesh=mesh,
        compiler_params=params,
        out_type=jax.ShapeDtypeStruct((NC * N_PAD, DH), jnp.float32),
        scratch_types=[
            pltpu.VMEM((J, CH), jnp.int32),
            pltpu.VMEM((J, CH), jnp.int32),
            pltpu.VMEM((2, KB, CH, DH), jnp.float32),
            pltpu.VMEM((RPT, DH), jnp.float32),
            pltpu.SemaphoreType.DMA,
            pltpu.SemaphoreType.DMA,
            pltpu.SemaphoreType.DMA,
            pltpu.VMEM_SHARED((N_PAD, DH), jnp.float32),
        ],
    )
    return deg, agg


def _in_body(degp_ref, x_ref, w_ref, h_ref, dinv_ref):
    deg = degp_ref[0] + degp_ref[1] + 1.0
    dinv = lax.rsqrt(deg)
    h = jnp.dot(x_ref[...], w_ref[...], preferred_element_type=jnp.float32)
    h_ref[0:N, :] = h * dinv
    h_ref[N:N_PAD, :] = jnp.zeros((N_PAD - N, DH), jnp.float32)
    dinv_ref[...] = dinv


def _mid_body(aggp_ref, h1_ref, dinv_ref, b_ref, w_ref, h_ref):
    su = aggp_ref[0, 0:N, :] + aggp_ref[1, 0:N, :] + h1_ref[0:N, :]
    z = jnp.maximum(su * dinv_ref[...] + b_ref[...], 0.0)
    h2 = jnp.dot(z, w_ref[...], preferred_element_type=jnp.float32)
    h_ref[0:N, :] = h2 * dinv_ref[...]
    h_ref[N:N_PAD, :] = jnp.zeros((N_PAD - N, DH), jnp.float32)


def _out_body(aggp_ref, h2_ref, dinv_ref, b_ref, out_ref):
    su = aggp_ref[0, 0:N, :] + aggp_ref[1, 0:N, :] + h2_ref[0:N, :]
    t = su * dinv_ref[...] + b_ref[...]
    m = jnp.max(t, axis=1, keepdims=True)
    lse = m + jnp.log(jnp.sum(jnp.exp(t - m), axis=1, keepdims=True))
    out_ref[...] = t - lse


_in_call = pl.pallas_call(
    _in_body,
    out_shape=[
        jax.ShapeDtypeStruct((N_PAD, DH), jnp.float32),
        jax.ShapeDtypeStruct((N, 1), jnp.float32),
    ],
)

_mid_call = pl.pallas_call(
    _mid_body,
    out_shape=jax.ShapeDtypeStruct((N_PAD, DH), jnp.float32),
)

_out_call = pl.pallas_call(
    _out_body,
    out_shape=jax.ShapeDtypeStruct((N, DH), jnp.float32),
)


@jax.jit
def kernel(x, edge_index, W1, b1, W2, b2):
    ei = edge_index.astype(jnp.int32)
    pad = jnp.full((E_PAD - E,), N, jnp.int32)
    src = jnp.concatenate([ei[0], pad]).reshape(NW, J, CH)
    dst = jnp.concatenate([ei[1], pad]).reshape(NW, J, CH)

    ones_c = jnp.ones((CH,), jnp.float32)
    zeros_r = jnp.zeros((RPT,), jnp.float32)
    zrows = jnp.zeros((RPT, DH), jnp.float32)

    deg_kernel, agg_kernel = _sc_kernels()
    degp = deg_kernel(dst, ones_c, zeros_r)
    degp = degp.reshape(NC, N_PAD)[:, :N, None]

    h1p, dinv = _in_call(degp, x, W1)
    agg1 = agg_kernel(h1p, src, dst, zrows).reshape(NC, N_PAD, DH)
    h2p = _mid_call(agg1, h1p, dinv, b1[None, :], W2)
    agg2 = agg_kernel(h2p, src, dst, zrows).reshape(NC, N_PAD, DH)
    return _out_call(agg2, h2p, dinv, b2[None, :])
